# Initial kernel scaffold; baseline (speedup 1.0000x reference)
import jax
import jax.numpy as jnp
from jax.experimental import pallas as pl


def kernel(v, edge_attr, W1, b1, bn_gamma, bn_beta, root_w, bias, edge_index):
    raise NotImplementedError("write your pallas kernel here")


# speedup vs baseline: 3.9305x; 3.9305x over previous
import functools

import jax
import jax.numpy as jnp
from jax import lax
from jax.experimental import pallas as pl
from jax.experimental.pallas import tpu as pltpu
from jax.experimental.pallas import tpu_sc as plsc

N = 10000
E = 160000
D = 32
K = 1024

NC = 2
NS = 16
NW = NC * NS
PER_W = E // NW
CH = 40
NCHUNK = PER_W // CH
MSTAGE = 1000
INNER = MSTAGE // CH
FIRE = 5
STRIPE = 624
TAIL = N - NS * STRIPE

_mesh = plsc.VectorSubcoreMesh(core_axis_name="c", subcore_axis_name="s")


def _prep_body(av_ref, bv_ref, w1_ref, b1_ref, g_ref, be_ref, coef_ref):
    a = av_ref[...]
    b = bv_ref[...]
    inv_e = 1.0 / E
    ma = jnp.sum(a) * inv_e
    mb = jnp.sum(b) * inv_e
    vaa = jnp.sum(a * a) * inv_e - ma * ma
    vbb = jnp.sum(b * b) * inv_e - mb * mb
    vab = jnp.sum(a * b) * inv_e - ma * mb
    w0 = w1_ref[0:1, :]
    w1 = w1_ref[1:2, :]
    mean = ma * w0 + mb * w1 + b1_ref[...]
    var = vaa * w0 * w0 + 2.0 * vab * w0 * w1 + vbb * w1 * w1
    s = g_ref[...] * lax.rsqrt(var + 1e-5)
    coef_ref[0:1, :] = s * w0
    coef_ref[1:2, :] = s * w1
    coef_ref[2:3, :] = (b1_ref[...] - mean) * s + be_ref[...]


def _prep(av, bv, w1, b1, g, be):
    return pl.pallas_call(
        _prep_body,
        out_shape=jax.ShapeDtypeStruct((3, K), jnp.float32),
    )(av, bv, w1, b1, g, be)


@functools.partial(
    pl.kernel,
    mesh=_mesh,
    out_type=jax.ShapeDtypeStruct((E, D), jnp.float32),
    scratch_types=[
        pltpu.VMEM((NCHUNK, CH), jnp.int32),
        pltpu.VMEM((MSTAGE, D), jnp.float32),
        pltpu.SemaphoreType.DMA,
    ],
    compiler_params=pltpu.CompilerParams(use_tc_tiling_on_sc=False),
)
def _gather(v_hbm, src_hbm, xj_hbm, idx_v, stage_v, sem):
    wid = lax.axis_index("s") * NC + lax.axis_index("c")
    pltpu.sync_copy(src_hbm.at[wid], idx_v)

    def macro(m, _):
        def fire(f, _):
            handles = []
            for b in range(FIRE):
                j = f * FIRE + b
                handles.append(pltpu.async_copy(
                    v_hbm.at[idx_v.at[m * INNER + j]],
                    stage_v.at[pl.ds(j * CH, CH)], sem))
            for h in handles:
                h.wait()
            return 0

        lax.fori_loop(0, INNER // FIRE, fire, 0)
        pltpu.sync_copy(stage_v,
                        xj_hbm.at[pl.ds(wid * PER_W + m * MSTAGE, MSTAGE)])
        return 0

    lax.fori_loop(0, PER_W // MSTAGE, macro, 0)


def _edge_body(ea_ref, xj_ref, coef_ref, r_ref, s_ref, msg_ref):
    a = ea_ref[:, 0:1]
    b = ea_ref[:, 1:2]
    g = a * coef_ref[0:1, :] + b * coef_ref[1:2, :] + coef_ref[2:3, :]
    h = jnp.tanh(g)
    xr = jnp.dot(xj_ref[...], r_ref[...], preferred_element_type=jnp.float32)
    msg_ref[...] = jnp.dot(xr * h, s_ref[...], preferred_element_type=jnp.float32)


def _edge(ea, xj, coef, r, s, block_e=640):
    grid = (E // block_e,)
    return pl.pallas_call(
        _edge_body,
        grid=grid,
        in_specs=[
            pl.BlockSpec((block_e, 2), lambda i: (i, 0)),
            pl.BlockSpec((block_e, D), lambda i: (i, 0)),
            pl.BlockSpec((3, K), lambda i: (0, 0)),
            pl.BlockSpec((D, K), lambda i: (0, 0)),
            pl.BlockSpec((K, D), lambda i: (0, 0)),
        ],
        out_specs=pl.BlockSpec((block_e, D), lambda i: (i, 0)),
        out_shape=jax.ShapeDtypeStruct((E, D), jnp.float32),
    )(ea, xj, coef, r, s)


@functools.partial(
    pl.kernel,
    mesh=_mesh,
    out_type=(
        jax.ShapeDtypeStruct((NC, N, D), jnp.float32),
        jax.ShapeDtypeStruct((NC, N, 16), jnp.float32),
    ),
    scratch_types=[
        pltpu.VMEM((NCHUNK, CH), jnp.int32),
        pltpu.VMEM((MSTAGE, D), jnp.float32),
        pltpu.VMEM((CH, 16), jnp.float32),
        pltpu.VMEM_SHARED((N, D), jnp.float32),
        pltpu.VMEM_SHARED((N, 16), jnp.float32),
    ],
    compiler_params=pltpu.CompilerParams(use_tc_tiling_on_sc=False),
)
def _scatter(msg_hbm, dst_hbm, z32_hbm, z16_hbm, ones_hbm,
             sums_hbm, cnt_hbm, idx_v, stage_v, ones_v, ssum, scnt):
    c = lax.axis_index("c")
    s = lax.axis_index("s")
    wid = s * NC + c

    pltpu.sync_copy(z32_hbm.at[pl.ds(s * STRIPE, STRIPE)],
                    ssum.at[pl.ds(s * STRIPE, STRIPE)])
    pltpu.sync_copy(z16_hbm.at[pl.ds(s * STRIPE, STRIPE)],
                    scnt.at[pl.ds(s * STRIPE, STRIPE)])

    @pl.when(s == NS - 1)
    def _zero_tail():
        pltpu.sync_copy(z32_hbm.at[pl.ds(NS * STRIPE, TAIL)],
                        ssum.at[pl.ds(NS * STRIPE, TAIL)])
        pltpu.sync_copy(z16_hbm.at[pl.ds(NS * STRIPE, TAIL)],
                        scnt.at[pl.ds(NS * STRIPE, TAIL)])

    pltpu.sync_copy(ones_hbm, ones_v)
    pltpu.sync_copy(dst_hbm.at[wid], idx_v)
    plsc.subcore_barrier()

    def outer(m, _):
        pltpu.sync_copy(msg_hbm.at[pl.ds(wid * PER_W + m * MSTAGE, MSTAGE)],
                        stage_v)

        def inner(j, _):
            jj = m * INNER + j
            pltpu.sync_copy(stage_v.at[pl.ds(j * CH, CH)],
                            ssum.at[idx_v.at[jj]], add=True)
            pltpu.sync_copy(ones_v, scnt.at[idx_v.at[jj]], add=True)
            return 0

        lax.fori_loop(0, INNER, inner, 0)
        return 0

    lax.fori_loop(0, PER_W // MSTAGE, outer, 0)
    plsc.subcore_barrier()

    pltpu.sync_copy(ssum.at[pl.ds(s * STRIPE, STRIPE)],
                    sums_hbm.at[c].at[pl.ds(s * STRIPE, STRIPE)])
    pltpu.sync_copy(scnt.at[pl.ds(s * STRIPE, STRIPE)],
                    cnt_hbm.at[c].at[pl.ds(s * STRIPE, STRIPE)])

    @pl.when(s == NS - 1)
    def _drain_tail():
        pltpu.sync_copy(ssum.at[pl.ds(NS * STRIPE, TAIL)],
                        sums_hbm.at[c].at[pl.ds(NS * STRIPE, TAIL)])
        pltpu.sync_copy(scnt.at[pl.ds(NS * STRIPE, TAIL)],
                        cnt_hbm.at[c].at[pl.ds(NS * STRIPE, TAIL)])


def _final_body(v_ref, s0_ref, s1_ref, c0_ref, c1_ref, rw_ref, bias_ref, o_ref):
    cnt = jnp.maximum(c0_ref[:, 0:1] + c1_ref[:, 0:1], 1.0)
    aggr = (s0_ref[...] + s1_ref[...]) / cnt
    root = jnp.dot(v_ref[...], rw_ref[...], preferred_element_type=jnp.float32)
    x = aggr + root + bias_ref[...]
    o_ref[...] = jnp.where(x >= 0.0, x, 0.01 * x)


def _final(v, s0, s1, c0, c1, rw, bias):
    return pl.pallas_call(
        _final_body,
        out_shape=jax.ShapeDtypeStruct((N, D), jnp.float32),
    )(v, s0, s1, c0, c1, rw, bias)


def kernel(v, edge_attr, W1, b1, bn_gamma, bn_beta, root_w, bias, edge_index):
    src = edge_index[0].reshape(NW, NCHUNK, CH)
    dst = edge_index[1].reshape(NW, NCHUNK, CH)
    ea_t = edge_attr.T
    av = ea_t[0].reshape(E // 128, 128)
    bv = ea_t[1].reshape(E // 128, 128)

    coef = _prep(av, bv, W1, b1.reshape(1, K),
                 bn_gamma.reshape(1, K), bn_beta.reshape(1, K))

    xj = _gather(v, src)

    k = jnp.arange(K, dtype=jnp.int32)
    r = (k[None, :] // D == jnp.arange(D, dtype=jnp.int32)[:, None])
    r = r.astype(jnp.float32)
    s = (k[:, None] % D == jnp.arange(D, dtype=jnp.int32)[None, :])
    s = s.astype(jnp.float32)

    msg = _edge(edge_attr, xj, coef, r, s)

    z32 = jnp.zeros((N, D), jnp.float32)
    z16 = jnp.zeros((N, 16), jnp.float32)
    ones = jnp.ones((CH, 16), jnp.float32)
    sums_p, cnt_p = _scatter(msg, dst, z32, z16, ones)

    return _final(v, sums_p[0], sums_p[1], cnt_p[0], cnt_p[1],
                  root_w, bias.reshape(1, D))


# speedup vs baseline: 4.0219x; 1.0232x over previous
import functools

import jax
import jax.numpy as jnp
from jax import lax
from jax.experimental import pallas as pl
from jax.experimental.pallas import tpu as pltpu
from jax.experimental.pallas import tpu_sc as plsc

N = 10000
E = 160000
D = 32
K = 1024

NC = 2
NS = 16
NW = NC * NS
PER_W = E // NW
CH = 40
NCHUNK = PER_W // CH
MSTAGE = 1000
INNER = MSTAGE // CH
FIRE = 5
STRIPE = 624
TAIL = N - NS * STRIPE

_mesh = plsc.VectorSubcoreMesh(core_axis_name="c", subcore_axis_name="s")


def _prep_body(x_ref, w1_ref, b1_ref, g_ref, be_ref, coef_ref):
    x = x_ref[...]
    lane = jax.lax.broadcasted_iota(jnp.int32, x.shape, 1)
    even = (lane % 2) == 0
    xz = jnp.where(even, x, 0.0)
    xr = jnp.where(even, jnp.roll(x, -1, axis=1), 0.0)
    inv_e = 1.0 / E
    st = jnp.sum(x) * inv_e
    ma = jnp.sum(xz) * inv_e
    mb = st - ma
    s2 = jnp.sum(x * x) * inv_e
    saa = jnp.sum(xz * xz) * inv_e
    vaa = saa - ma * ma
    vbb = (s2 - saa) - mb * mb
    vab = jnp.sum(xz * xr) * inv_e - ma * mb
    w0 = w1_ref[0:1, :]
    w1 = w1_ref[1:2, :]
    mean = ma * w0 + mb * w1 + b1_ref[...]
    var = vaa * w0 * w0 + 2.0 * vab * w0 * w1 + vbb * w1 * w1
    s = g_ref[...] * lax.rsqrt(var + 1e-5)
    coef_ref[0:1, :] = s * w0
    coef_ref[1:2, :] = s * w1
    coef_ref[2:3, :] = (b1_ref[...] - mean) * s + be_ref[...]


def _prep(x, w1, b1, g, be):
    return pl.pallas_call(
        _prep_body,
        out_shape=jax.ShapeDtypeStruct((3, K), jnp.float32),
    )(x, w1, b1, g, be)


@functools.partial(
    pl.kernel,
    mesh=_mesh,
    out_type=jax.ShapeDtypeStruct((E, D), jnp.float32),
    scratch_types=[
        pltpu.VMEM((NCHUNK, CH), jnp.int32),
        pltpu.VMEM((MSTAGE, D), jnp.float32),
        pltpu.SemaphoreType.DMA,
    ],
    compiler_params=pltpu.CompilerParams(use_tc_tiling_on_sc=False),
)
def _gather(v_hbm, src_hbm, xj_hbm, idx_v, stage_v, sem):
    wid = lax.axis_index("s") * NC + lax.axis_index("c")
    pltpu.sync_copy(src_hbm.at[wid], idx_v)

    def macro(m, _):
        def fire(f, _):
            handles = []
            for b in range(FIRE):
                j = f * FIRE + b
                handles.append(pltpu.async_copy(
                    v_hbm.at[idx_v.at[m * INNER + j]],
                    stage_v.at[pl.ds(j * CH, CH)], sem))
            for h in handles:
                h.wait()
            return 0

        lax.fori_loop(0, INNER // FIRE, fire, 0)
        pltpu.sync_copy(stage_v,
                        xj_hbm.at[pl.ds(wid * PER_W + m * MSTAGE, MSTAGE)])
        return 0

    lax.fori_loop(0, PER_W // MSTAGE, macro, 0)


def _edge_body(ea_ref, xj_ref, coef_ref, r_ref, s_ref, msg_ref):
    a = ea_ref[:, 0:1]
    b = ea_ref[:, 1:2]
    g = a * coef_ref[0:1, :] + b * coef_ref[1:2, :] + coef_ref[2:3, :]
    h = jnp.tanh(g)
    xr = jnp.dot(xj_ref[...], r_ref[...], preferred_element_type=jnp.float32)
    msg_ref[...] = jnp.dot(xr * h, s_ref[...],
                           preferred_element_type=jnp.float32)


def _edge(ea, xj, coef, r, s, block_e=1280):
    grid = (E // block_e,)
    return pl.pallas_call(
        _edge_body,
        grid=grid,
        in_specs=[
            pl.BlockSpec((block_e, 2), lambda i: (i, 0)),
            pl.BlockSpec((block_e, D), lambda i: (i, 0)),
            pl.BlockSpec((3, K), lambda i: (0, 0)),
            pl.BlockSpec((D, K), lambda i: (0, 0)),
            pl.BlockSpec((K, D), lambda i: (0, 0)),
        ],
        out_specs=pl.BlockSpec((block_e, D), lambda i: (i, 0)),
        out_shape=jax.ShapeDtypeStruct((E, D), jnp.float32),
    )(ea, xj, coef, r, s)


@functools.partial(
    pl.kernel,
    mesh=_mesh,
    out_type=(
        jax.ShapeDtypeStruct((NC, N, D), jnp.float32),
        jax.ShapeDtypeStruct((NC, N, 16), jnp.float32),
    ),
    scratch_types=[
        pltpu.VMEM((NCHUNK, CH), jnp.int32),
        pltpu.VMEM((MSTAGE, D), jnp.float32),
        pltpu.VMEM((CH, 16), jnp.float32),
        pltpu.VMEM_SHARED((N, D), jnp.float32),
        pltpu.VMEM_SHARED((N, 16), jnp.float32),
    ],
    compiler_params=pltpu.CompilerParams(use_tc_tiling_on_sc=False),
)
def _scatter(msg_hbm, dst_hbm, z32_hbm, z16_hbm, ones_hbm,
             sums_hbm, cnt_hbm, idx_v, stage_v, ones_v, ssum, scnt):
    c = lax.axis_index("c")
    s = lax.axis_index("s")
    wid = s * NC + c

    pltpu.sync_copy(z32_hbm.at[pl.ds(s * STRIPE, STRIPE)],
                    ssum.at[pl.ds(s * STRIPE, STRIPE)])
    pltpu.sync_copy(z16_hbm.at[pl.ds(s * STRIPE, STRIPE)],
                    scnt.at[pl.ds(s * STRIPE, STRIPE)])

    @pl.when(s == NS - 1)
    def _zero_tail():
        pltpu.sync_copy(z32_hbm.at[pl.ds(NS * STRIPE, TAIL)],
                        ssum.at[pl.ds(NS * STRIPE, TAIL)])
        pltpu.sync_copy(z16_hbm.at[pl.ds(NS * STRIPE, TAIL)],
                        scnt.at[pl.ds(NS * STRIPE, TAIL)])

    pltpu.sync_copy(ones_hbm, ones_v)
    pltpu.sync_copy(dst_hbm.at[wid], idx_v)
    plsc.subcore_barrier()

    def outer(m, _):
        pltpu.sync_copy(msg_hbm.at[pl.ds(wid * PER_W + m * MSTAGE, MSTAGE)],
                        stage_v)

        def inner(j, _):
            jj = m * INNER + j
            pltpu.sync_copy(stage_v.at[pl.ds(j * CH, CH)],
                            ssum.at[idx_v.at[jj]], add=True)
            pltpu.sync_copy(ones_v, scnt.at[idx_v.at[jj]], add=True)
            return 0

        lax.fori_loop(0, INNER, inner, 0)
        return 0

    lax.fori_loop(0, PER_W // MSTAGE, outer, 0)
    plsc.subcore_barrier()

    pltpu.sync_copy(ssum.at[pl.ds(s * STRIPE, STRIPE)],
                    sums_hbm.at[c].at[pl.ds(s * STRIPE, STRIPE)])
    pltpu.sync_copy(scnt.at[pl.ds(s * STRIPE, STRIPE)],
                    cnt_hbm.at[c].at[pl.ds(s * STRIPE, STRIPE)])

    @pl.when(s == NS - 1)
    def _drain_tail():
        pltpu.sync_copy(ssum.at[pl.ds(NS * STRIPE, TAIL)],
                        sums_hbm.at[c].at[pl.ds(NS * STRIPE, TAIL)])
        pltpu.sync_copy(scnt.at[pl.ds(NS * STRIPE, TAIL)],
                        cnt_hbm.at[c].at[pl.ds(NS * STRIPE, TAIL)])


def _final_body(v_ref, s0_ref, s1_ref, c0_ref, c1_ref, rw_ref, bias_ref, o_ref):
    cnt = jnp.maximum(c0_ref[:, 0:1] + c1_ref[:, 0:1], 1.0)
    aggr = (s0_ref[...] + s1_ref[...]) / cnt
    root = jnp.dot(v_ref[...], rw_ref[...], preferred_element_type=jnp.float32)
    x = aggr + root + bias_ref[...]
    o_ref[...] = jnp.where(x >= 0.0, x, 0.01 * x)


def _final(v, s0, s1, c0, c1, rw, bias):
    return pl.pallas_call(
        _final_body,
        out_shape=jax.ShapeDtypeStruct((N, D), jnp.float32),
    )(v, s0, s1, c0, c1, rw, bias)


def kernel(v, edge_attr, W1, b1, bn_gamma, bn_beta, root_w, bias, edge_index):
    src = edge_index[0].reshape(NW, NCHUNK, CH)
    dst = edge_index[1].reshape(NW, NCHUNK, CH)
    eav = edge_attr.reshape(E // 64, 128)

    coef = _prep(eav, W1, b1.reshape(1, K),
                 bn_gamma.reshape(1, K), bn_beta.reshape(1, K))

    xj = _gather(v, src)

    k = jnp.arange(K, dtype=jnp.int32)
    r = (k[None, :] // D == jnp.arange(D, dtype=jnp.int32)[:, None])
    r = r.astype(jnp.float32)
    s = (k[:, None] % D == jnp.arange(D, dtype=jnp.int32)[None, :])
    s = s.astype(jnp.float32)

    msg = _edge(edge_attr, xj, coef, r, s)

    z32 = jnp.zeros((N, D), jnp.float32)
    z16 = jnp.zeros((N, 16), jnp.float32)
    ones = jnp.ones((CH, 16), jnp.float32)
    sums_p, cnt_p = _scatter(msg, dst, z32, z16, ones)

    return _final(v, sums_p[0], sums_p[1], cnt_p[0], cnt_p[1],
                  root_w, bias.reshape(1, D))
